# baseline (speedup 1.0000x reference)
import functools

import jax
import jax.numpy as jnp
from jax import lax
from jax.experimental import pallas as pl
from jax.experimental.pallas import tpu as pltpu
from jax.experimental.pallas import tpu_sc as plsc

NSEG = 256
D = 128
N_ROWS = 100000
NC = 2
NS = 16
NW = NC * NS
CH = 128
N_FULL = 24
T_BIG = 56
T_SMALL = 48
BIG_WORKERS = 20
NBUF = 4
VEC_CHUNKS = frozenset(j for j in range(N_FULL) if j % 5 in (1, 3))

_mesh = plsc.VectorSubcoreMesh(core_axis_name="c", subcore_axis_name="s")


@functools.partial(
    pl.kernel,
    mesh=_mesh,
    compiler_params=pltpu.CompilerParams(needs_layout_passes=False),
    out_type=jax.ShapeDtypeStruct((NC, NSEG, D), jnp.float32),
    scratch_types=(
        [pltpu.VMEM((CH,), jnp.int32) for _ in range(NBUF)]
        + [pltpu.VMEM((CH, D), jnp.float32) for _ in range(NBUF)]
        + [
            pltpu.VMEM((NSEG, D), jnp.float32),
            pltpu.VMEM((T_BIG,), jnp.int32),
            pltpu.VMEM((T_SMALL,), jnp.int32),
            pltpu.VMEM((T_BIG, D), jnp.float32),
            pltpu.VMEM((T_SMALL, D), jnp.float32),
            pltpu.VMEM((NSEG // 2,), jnp.int32),
            pltpu.VMEM((NSEG // 2,), jnp.int32),
            pltpu.VMEM((NS, D), jnp.float32),
            pltpu.VMEM_SHARED((NSEG, D), jnp.float32),
        ]
        + [pltpu.SemaphoreType.DMA for _ in range(3 * NBUF + 2)]
    ),
)
def _sc_partials(feat_hbm, ids_hbm, out_hbm, *scratch):
    ids_bufs = scratch[0:NBUF]
    row_bufs = scratch[NBUF:2 * NBUF]
    (acc_l, ids_tb, ids_ts, rows_tb, rows_ts, idn_lo, idn_hi, zbuf,
     acc) = scratch[2 * NBUF:2 * NBUF + 9]
    sems = scratch[2 * NBUF + 9:]
    sem_i = sems[0:NBUF]
    sem_r = sems[NBUF:2 * NBUF]
    sem_s = sems[2 * NBUF:3 * NBUF]
    sem_t = sems[3 * NBUF]
    sem_m = sems[3 * NBUF + 1]

    c = lax.axis_index("c")
    s = lax.axis_index("s")
    wid = s * NC + c
    base = pl.multiple_of(3120 * wid + 8 * jnp.minimum(wid, BIG_WORKERS), 8)

    loads = {}

    def start_load(j):
        p = j % NBUF
        off = pl.multiple_of(base + j * CH, 8)
        ci = pltpu.make_async_copy(ids_hbm.at[pl.ds(off, CH)], ids_bufs[p], sem_i[p])
        cr = pltpu.make_async_copy(feat_hbm.at[pl.ds(off, CH), :], row_bufs[p], sem_r[p])
        ci.start()
        cr.start()
        loads[j] = (ci, cr)

    start_load(0)

    zero = jnp.zeros((16,), jnp.float32)
    iota = lax.iota(jnp.int32, 16)
    for i in range(NS):
        for j in range(D // 16):
            zbuf[i, pl.ds(j * 16, 16)] = zero
    pltpu.sync_copy(zbuf, acc.at[pl.ds(s * NS, NS), :])
    plsc.subcore_barrier()

    for j in range(1, NBUF - 1):
        start_load(j)

    toff = pl.multiple_of(base + N_FULL * CH, 8)
    t_ib = pltpu.make_async_copy(ids_hbm.at[pl.ds(toff, T_BIG)], ids_tb, sem_t)
    t_rb = pltpu.make_async_copy(feat_hbm.at[pl.ds(toff, T_BIG), :], rows_tb, sem_t)
    t_is = pltpu.make_async_copy(ids_hbm.at[pl.ds(toff, T_SMALL)], ids_ts, sem_t)
    t_rs = pltpu.make_async_copy(feat_hbm.at[pl.ds(toff, T_SMALL), :], rows_ts, sem_t)

    @pl.when(wid < BIG_WORKERS)
    def _start_big_tail():
        t_ib.start()
        t_rb.start()

    @pl.when(wid >= BIG_WORKERS)
    def _start_small_tail():
        t_is.start()
        t_rs.start()

    for k in range(NSEG // 32):
        idn_lo[pl.ds(k * 16, 16)] = iota + (k * 16)
        idn_hi[pl.ds(k * 16, 16)] = iota + (NSEG // 2 + k * 16)

    def zero_acc_rows(i, carry):
        for r in range(8):
            for jc in range(D // 16):
                acc_l[i * 8 + r, pl.ds(jc * 16, 16)] = zero
        return carry

    lax.fori_loop(0, NSEG // 8, zero_acc_rows, 0)

    col_idx = [iota + (jc * 16) for jc in range(D // 16)]
    pending = {}

    for j in range(N_FULL):
        p = j % NBUF
        ci, cr = loads.pop(j)
        ci.wait()
        cr.wait()
        if j in VEC_CHUNKS:
            rows_v = row_bufs[p]
            ids_v = ids_bufs[p]

            def accum_block(i, carry, rows_v=rows_v, ids_v=ids_v):
                ids16 = ids_v[pl.ds(i * 16, 16)]
                for r in range(16):
                    rid = ids16.at[jnp.full((16,), r, jnp.int32)].get(
                        mode="promise_in_bounds")
                    for jc in range(D // 16):
                        vals = rows_v[i * 16 + r, pl.ds(jc * 16, 16)]
                        plsc.addupdate_scatter(acc_l, [rid, col_idx[jc]], vals)
                return carry

            lax.fori_loop(0, CH // 16, accum_block, 0)
        else:
            for jj in sorted(pending):
                pending.pop(jj).wait()
            sc = pltpu.make_async_copy(row_bufs[p], acc.at[ids_bufs[p]], sem_s[p])
            sc.start(add=True)
            pending[j] = sc
        nxt = j + NBUF - 1
        if nxt < N_FULL:
            prev = nxt - NBUF
            if prev in pending:
                pending.pop(prev).wait()
            start_load(nxt)
    for jj in sorted(pending):
        pending.pop(jj).wait()

    @pl.when(wid < BIG_WORKERS)
    def _big_tail():
        t_ib.wait()
        t_rb.wait()
        pltpu.sync_copy(rows_tb, acc.at[ids_tb], add=True)

    @pl.when(wid >= BIG_WORKERS)
    def _small_tail():
        t_is.wait()
        t_rs.wait()
        pltpu.sync_copy(rows_ts, acc.at[ids_ts], add=True)

    m_lo = pltpu.make_async_copy(acc_l.at[pl.ds(0, NSEG // 2), :],
                                 acc.at[idn_lo], sem_m)
    m_lo.start(add=True)
    m_hi = pltpu.make_async_copy(acc_l.at[pl.ds(NSEG // 2, NSEG // 2), :],
                                 acc.at[idn_hi], sem_m)
    m_hi.start(add=True)
    m_lo.wait()
    m_hi.wait()

    plsc.subcore_barrier()
    pltpu.sync_copy(
        acc.at[pl.ds(s * NS, NS), :],
        out_hbm.at[c, pl.ds(s * NS, NS), :],
    )


def _combine(partials):
    def body(p_ref, o_ref):
        o_ref[...] = p_ref[0, :, :] + p_ref[1, :, :]

    return pl.pallas_call(
        body,
        out_shape=jax.ShapeDtypeStruct((NSEG, D), jnp.float32),
    )(partials)


def kernel(feat, segment_ids):
    partials = _sc_partials(feat, segment_ids.astype(jnp.int32))
    return _combine(partials)


# speedup vs baseline: 1.2023x; 1.2023x over previous
import functools

import jax
import jax.numpy as jnp
from jax import lax
from jax.experimental import pallas as pl
from jax.experimental.pallas import tpu as pltpu
from jax.experimental.pallas import tpu_sc as plsc

NSEG = 256
D = 128
N_ROWS = 100000
NC = 2
NS = 16
NW = NC * NS
CH = 128
N_FULL = 24
T_BIG = 56
T_SMALL = 48
BIG_WORKERS = 20
NBUF = 4

_mesh = plsc.VectorSubcoreMesh(core_axis_name="c", subcore_axis_name="s")


@functools.partial(
    pl.kernel,
    mesh=_mesh,
    out_type=jax.ShapeDtypeStruct((NC, NSEG, D), jnp.float32),
    scratch_types=(
        [pltpu.VMEM((CH,), jnp.int32) for _ in range(NBUF)]
        + [pltpu.VMEM((CH, D), jnp.float32) for _ in range(NBUF)]
        + [
            pltpu.VMEM((T_BIG,), jnp.int32),
            pltpu.VMEM((T_SMALL,), jnp.int32),
            pltpu.VMEM((T_BIG, D), jnp.float32),
            pltpu.VMEM((T_SMALL, D), jnp.float32),
            pltpu.VMEM((NS, D), jnp.float32),
            pltpu.VMEM_SHARED((NSEG, D), jnp.float32),
        ]
        + [pltpu.SemaphoreType.DMA for _ in range(3 * NBUF + 1)]
    ),
)
def _sc_partials(feat_hbm, ids_hbm, out_hbm, *scratch):
    ids_bufs = scratch[0:NBUF]
    row_bufs = scratch[NBUF:2 * NBUF]
    ids_tb, ids_ts, rows_tb, rows_ts, zbuf, acc = scratch[2 * NBUF:2 * NBUF + 6]
    sems = scratch[2 * NBUF + 6:]
    sem_i = sems[0:NBUF]
    sem_r = sems[NBUF:2 * NBUF]
    sem_s = sems[2 * NBUF:3 * NBUF]
    sem_t = sems[3 * NBUF]

    c = lax.axis_index("c")
    s = lax.axis_index("s")
    wid = s * NC + c
    base = pl.multiple_of(3120 * wid + 8 * jnp.minimum(wid, BIG_WORKERS), 8)

    loads = {}

    def start_load(j):
        p = j % NBUF
        off = pl.multiple_of(base + j * CH, 8)
        ci = pltpu.make_async_copy(ids_hbm.at[pl.ds(off, CH)], ids_bufs[p], sem_i[p])
        cr = pltpu.make_async_copy(feat_hbm.at[pl.ds(off, CH), :], row_bufs[p], sem_r[p])
        ci.start()
        cr.start()
        loads[j] = (ci, cr)

    start_load(0)

    zero = jnp.zeros((16,), jnp.float32)
    for i in range(NS):
        for j in range(D // 16):
            zbuf[i, pl.ds(j * 16, 16)] = zero
    pltpu.sync_copy(zbuf, acc.at[pl.ds(s * NS, NS), :])
    plsc.subcore_barrier()

    for j in range(1, NBUF - 1):
        start_load(j)

    toff = pl.multiple_of(base + N_FULL * CH, 8)
    t_ib = pltpu.make_async_copy(ids_hbm.at[pl.ds(toff, T_BIG)], ids_tb, sem_t)
    t_rb = pltpu.make_async_copy(feat_hbm.at[pl.ds(toff, T_BIG), :], rows_tb, sem_t)
    t_is = pltpu.make_async_copy(ids_hbm.at[pl.ds(toff, T_SMALL)], ids_ts, sem_t)
    t_rs = pltpu.make_async_copy(feat_hbm.at[pl.ds(toff, T_SMALL), :], rows_ts, sem_t)

    @pl.when(wid < BIG_WORKERS)
    def _start_big_tail():
        t_ib.start()
        t_rb.start()

    @pl.when(wid >= BIG_WORKERS)
    def _start_small_tail():
        t_is.start()
        t_rs.start()

    scats = {}
    for j in range(N_FULL):
        p = j % NBUF
        ci, cr = loads.pop(j)
        ci.wait()
        cr.wait()
        if j >= 1:
            scats.pop(j - 1).wait()
        sc = pltpu.make_async_copy(row_bufs[p], acc.at[ids_bufs[p]], sem_s[p])
        sc.start(add=True)
        scats[j] = sc
        nxt = j + NBUF - 1
        if nxt < N_FULL:
            start_load(nxt)
    scats.pop(N_FULL - 1).wait()

    @pl.when(wid < BIG_WORKERS)
    def _big_tail():
        t_ib.wait()
        t_rb.wait()
        pltpu.sync_copy(rows_tb, acc.at[ids_tb], add=True)

    @pl.when(wid >= BIG_WORKERS)
    def _small_tail():
        t_is.wait()
        t_rs.wait()
        pltpu.sync_copy(rows_ts, acc.at[ids_ts], add=True)

    plsc.subcore_barrier()
    pltpu.sync_copy(
        acc.at[pl.ds(s * NS, NS), :],
        out_hbm.at[c, pl.ds(s * NS, NS), :],
    )


def _combine(partials):
    def body(p_ref, o_ref):
        o_ref[...] = p_ref[0, :, :] + p_ref[1, :, :]

    return pl.pallas_call(
        body,
        out_shape=jax.ShapeDtypeStruct((NSEG, D), jnp.float32),
    )(partials)


def kernel(feat, segment_ids):
    partials = _sc_partials(feat, segment_ids.astype(jnp.int32))
    return _combine(partials)
